# baseline (speedup 1.0000x reference)
import functools

import jax
import jax.numpy as jnp
from jax import lax
from jax.experimental import pallas as pl
from jax.experimental.pallas import tpu as pltpu
from jax.experimental.pallas import tpu_sc as plsc

IMG = 224; PATCH = 16; C_IN = 3; EMBED = 512; NEXP = 10; TOPK = 2; NHEAD = 8; BATCH = 8
NTOK = (IMG // PATCH) ** 2
PDIM = PATCH * PATCH * C_IN
HDIM = 4 * EMBED
HD = EMBED // NHEAD
R = BATCH * NTOK
TTILE = 224
NTILE = R // TTILE

A = R * TOPK
TILE = 128
GTOT = A // TILE + NEXP
NPAD = GTOT * TILE
FPAD = 1792
EPAD = 16384
NSC = 32

_F32 = jnp.float32


def _dot(a, b, dims):
    return jax.lax.dot_general(a, b, (dims, ((), ())),
                               preferred_element_type=_F32)


def _ln_rows(x, g, b, eps=1e-5):
    m = jnp.mean(x, axis=-1, keepdims=True)
    v = jnp.mean((x - m) ** 2, axis=-1, keepdims=True)
    return (x - m) / jnp.sqrt(v + eps) * g + b


def _embed_attn_body(xp_ref, wp_ref, bp_ref, g1_ref, b1_ref,
                     wq_ref, wk_ref, wv_ref, wo_ref, bo_ref, pos_ref, t_ref):
    x = xp_ref[0]
    t0 = _dot(x, wp_ref[...], ((1,), (1,))) + bp_ref[...]
    ln = _ln_rows(t0, g1_ref[...], b1_ref[...])
    q = _dot(ln, wq_ref[...], ((1,), (1,)))
    k = _dot(ln, wk_ref[...], ((1,), (1,)))
    v = _dot(ln, wv_ref[...], ((1,), (1,)))
    heads = []
    for h in range(NHEAD):
        sl = slice(h * HD, (h + 1) * HD)
        att = _dot(q[:, sl], k[:, sl], ((1,), (1,))) * (HD ** -0.5)
        att = jax.nn.softmax(att, axis=-1)
        heads.append(_dot(att, v[:, sl], ((1,), (0,))))
    o = jnp.concatenate(heads, axis=-1)
    o = _dot(o, wo_ref[...], ((1,), (1,))) + bo_ref[...]
    t_ref[0] = t0 + o + pos_ref[0]


def _embed_attn(xp, p):
    return pl.pallas_call(
        _embed_attn_body,
        grid=(BATCH,),
        in_specs=[
            pl.BlockSpec((1, NTOK, PDIM), lambda b: (b, 0, 0)),
            pl.BlockSpec((EMBED, PDIM), lambda b: (0, 0)),
            pl.BlockSpec((1, EMBED), lambda b: (0, 0)),
            pl.BlockSpec((1, EMBED), lambda b: (0, 0)),
            pl.BlockSpec((1, EMBED), lambda b: (0, 0)),
            pl.BlockSpec((EMBED, EMBED), lambda b: (0, 0)),
            pl.BlockSpec((EMBED, EMBED), lambda b: (0, 0)),
            pl.BlockSpec((EMBED, EMBED), lambda b: (0, 0)),
            pl.BlockSpec((EMBED, EMBED), lambda b: (0, 0)),
            pl.BlockSpec((1, EMBED), lambda b: (0, 0)),
            pl.BlockSpec((1, NTOK, EMBED), lambda b: (0, 0, 0)),
        ],
        out_specs=pl.BlockSpec((1, NTOK, EMBED), lambda b: (b, 0, 0)),
        out_shape=jax.ShapeDtypeStruct((BATCH, NTOK, EMBED), _F32),
    )(xp, p['Wp'], p['bp'].reshape(1, -1), p['g1'].reshape(1, -1),
      p['bln1'].reshape(1, -1), p['Wq'], p['Wk'], p['Wv'], p['Wo'],
      p['bo'].reshape(1, -1), p['pos'])


def _router_body(t_ref, g_ref, b_ref, wt_ref, bt_ref, wn_ref, bn_ref,
                 noise_ref, xln_ref, gate_ref):
    x = _ln_rows(t_ref[...], g_ref[...], b_ref[...])
    logits = _dot(x, wt_ref[...], ((1,), (1,))) + bt_ref[...]
    nl = _dot(x, wn_ref[...], ((1,), (1,))) + bn_ref[...]
    noisy = logits + noise_ref[...] * jax.nn.softplus(nl)
    m1 = jnp.max(noisy, axis=-1, keepdims=True)
    ninf = jnp.float32(-jnp.inf)
    m2 = jnp.max(jnp.where(noisy == m1, ninf, noisy), axis=-1, keepdims=True)
    sel = noisy >= m2
    e = jnp.where(sel, jnp.exp(noisy - m1), 0.0)
    gate_ref[...] = e / jnp.sum(e, axis=-1, keepdims=True)
    xln_ref[...] = x


def _router(t_flat, mp, g, b, noise):
    full = lambda *s: pl.BlockSpec(s, lambda: tuple(0 for _ in s))
    return pl.pallas_call(
        _router_body,
        in_specs=[
            full(R, EMBED), full(1, EMBED), full(1, EMBED),
            full(NEXP, EMBED), full(1, NEXP),
            full(NEXP, EMBED), full(1, NEXP), full(R, NEXP),
        ],
        out_specs=[full(R, EMBED), full(R, NEXP)],
        out_shape=[jax.ShapeDtypeStruct((R, EMBED), _F32),
                   jax.ShapeDtypeStruct((R, NEXP), _F32)],
    )(t_flat, g.reshape(1, -1), b.reshape(1, -1),
      mp['Wt'], mp['bt'].reshape(1, -1), mp['Wn'], mp['bn'].reshape(1, -1),
      noise)


def _route_meta(topi, gating):
    i32 = jnp.int32
    token = (jnp.arange(A, dtype=i32) // TOPK)
    expert = topi.reshape(A).astype(i32)
    key = expert * R + token
    order = jnp.argsort(key)
    e_s, t_s, key_s = expert[order], token[order], key[order]
    counts = jnp.zeros((NEXP,), i32).at[expert].add(1)
    tiles = jnp.maximum((counts + TILE - 1) // TILE, 1)
    cum_tiles = jnp.cumsum(tiles)
    rowstart = (cum_tiles - tiles) * TILE
    gfirst = jnp.cumsum(counts) - counts
    pos_s = rowstart[e_s] + jnp.arange(A, dtype=i32) - gfirst[e_s]
    row_token = (jnp.arange(NPAD, dtype=i32) % R).at[pos_s].set(t_s)
    row_gate = jnp.zeros((NPAD,), _F32).at[pos_s].set(gating[t_s, e_s])
    covered = jnp.zeros((NPAD,), i32).at[pos_s].set(1)
    pad_rows = jnp.argsort(covered)[:NPAD - A].astype(i32)
    tile_expert = jnp.minimum(
        jnp.searchsorted(cum_tiles, jnp.arange(GTOT, dtype=i32), side='right'),
        NEXP - 1).astype(i32)
    pos_a = jnp.zeros((A,), i32).at[order].set(pos_s)
    pos0p = jnp.concatenate([pos_a[0::2], pad_rows[:FPAD - R]])
    pos1p = jnp.concatenate([pos_a[1::2], pad_rows[FPAD - R:2 * (FPAD - R)]])
    d = jnp.arange(EPAD, dtype=i32)
    j = jnp.minimum(jnp.searchsorted(key_s, jnp.minimum(d, NEXP * R - 1)), A - 1)
    exists = (key_s[j] == d) & (d < NEXP * R)
    e_map = jnp.where(exists, pos_s[j], pad_rows[d % (NPAD - A)])
    return row_token, row_gate, tile_expert, pos0p, pos1p, e_map


SUB = 16
SCH = 128 // SUB


def _sc_gather(table, idx, n_rows):
    d = table.shape[1]
    nsup = n_rows // 128
    full, rem = divmod(nsup, NSC)
    mesh = plsc.VectorSubcoreMesh(core_axis_name="c", subcore_axis_name="s")

    @functools.partial(
        pl.kernel, mesh=mesh,
        out_type=jax.ShapeDtypeStruct((nsup, 128, d), _F32),
        scratch_types=[pltpu.VMEM((128,), jnp.int32),
                       pltpu.VMEM((128, d), _F32)]
                      + [pltpu.SemaphoreType.DMA] * SUB,
    )
    def k(table_hbm, idx_hbm, out_hbm, idx_v, rows_v, *sems):
        wid = lax.axis_index("s") * 2 + lax.axis_index("c")

        def do_superchunk(s):
            pltpu.sync_copy(idx_hbm.at[s], idx_v)
            cps = [pltpu.async_copy(
                table_hbm.at[idx_v.at[pl.ds(i * SCH, SCH)]],
                rows_v.at[pl.ds(i * SCH, SCH)], sems[i])
                for i in range(SUB)]
            for cp in cps:
                cp.wait()
            pltpu.sync_copy(rows_v, out_hbm.at[s])

        for r in range(full):
            do_superchunk(r * NSC + wid)
        if rem:
            @pl.when(wid < rem)
            def _():
                do_superchunk(full * NSC + wid)

    out = k(table, idx.reshape(nsup, 128))
    return out.reshape(n_rows, d)


def _ffn_body(se_ref, x_ref, w1_ref, b1_ref, w2_ref, b2_ref, g_ref, o_ref):
    del se_ref
    x = x_ref[...]
    h = jnp.maximum(_dot(x, w1_ref[0], ((1,), (1,))) + b1_ref[0], 0.0)
    o = _dot(h, w2_ref[0], ((1,), (1,))) + b2_ref[0]
    o_ref[...] = o * g_ref[0, 0][:, None]


def _moe_ffn(xs, tile_expert, row_gate, mp):
    grid_spec = pltpu.PrefetchScalarGridSpec(
        num_scalar_prefetch=1,
        grid=(GTOT,),
        in_specs=[
            pl.BlockSpec((TILE, EMBED), lambda g, se: (g, 0)),
            pl.BlockSpec((1, HDIM, EMBED), lambda g, se: (se[g], 0, 0)),
            pl.BlockSpec((1, 1, HDIM), lambda g, se: (se[g], 0, 0)),
            pl.BlockSpec((1, EMBED, HDIM), lambda g, se: (se[g], 0, 0)),
            pl.BlockSpec((1, 1, EMBED), lambda g, se: (se[g], 0, 0)),
            pl.BlockSpec((1, 1, TILE), lambda g, se: (g, 0, 0)),
        ],
        out_specs=pl.BlockSpec((TILE, EMBED), lambda g, se: (g, 0)),
    )
    return pl.pallas_call(
        _ffn_body,
        grid_spec=grid_spec,
        out_shape=jax.ShapeDtypeStruct((NPAD, EMBED), _F32),
    )(tile_expert, xs, mp['W1'], mp['b1'].reshape(NEXP, 1, HDIM),
      mp['W2'], mp['b2'].reshape(NEXP, 1, EMBED),
      row_gate.reshape(GTOT, 1, TILE))


def _moe_sparse(xln, gating, mp):
    topi = lax.top_k(gating, TOPK)[1]
    row_token, row_gate, tile_expert, pos0p, pos1p, e_map = _route_meta(
        topi, gating)
    xs = _sc_gather(xln, row_token, NPAD)
    wrows = _moe_ffn(xs, tile_expert, row_gate, mp)
    f_ab = _sc_gather(wrows, jnp.concatenate([pos0p, pos1p]), 2 * FPAD)
    e_pad = _sc_gather(wrows, e_map, EPAD)
    return f_ab[:FPAD], f_ab[FPAD:], e_pad


def _combine_body(a1_ref, b1_ref, a2_ref, b2_ref, f1_ref, f2_ref):
    f1_ref[...] = a1_ref[...] + b1_ref[...]
    f2_ref[...] = a2_ref[...] + b2_ref[...]


def _combine(a1, b1, a2, b2):
    full = lambda *s: pl.BlockSpec(s, lambda: tuple(0 for _ in s))
    return pl.pallas_call(
        _combine_body,
        in_specs=[full(FPAD, EMBED)] * 4,
        out_specs=[full(FPAD, EMBED)] * 2,
        out_shape=[jax.ShapeDtypeStruct((FPAD, EMBED), _F32)] * 2,
    )(a1, b1, a2, b2)


def _head_body(f2_ref, wc_ref, bc_ref, feat_ref, cls_ref):
    feat = jnp.mean(f2_ref[...], axis=1)
    feat_ref[...] = feat
    cls_ref[...] = _dot(feat, wc_ref[...], ((1,), (1,))) + bc_ref[...]


def _head(f2, wc, bc):
    full = lambda *s: pl.BlockSpec(s, lambda: tuple(0 for _ in s))
    return pl.pallas_call(
        _head_body,
        in_specs=[full(BATCH, NTOK, EMBED), full(NEXP, EMBED), full(1, NEXP)],
        out_specs=[full(BATCH, EMBED), full(BATCH, NEXP)],
        out_shape=[jax.ShapeDtypeStruct((BATCH, EMBED), _F32),
                   jax.ShapeDtypeStruct((BATCH, NEXP), _F32)],
    )(f2, wc, bc.reshape(1, -1))


def kernel(x, params):
    b, c, h, w = x.shape
    xp = x.reshape(b, c, h // PATCH, PATCH, w // PATCH, PATCH)
    xp = xp.transpose(0, 1, 2, 4, 3, 5).reshape(b, c, -1, PATCH * PATCH)
    xp = xp.transpose(0, 2, 1, 3).reshape(b, -1, PDIM)

    t = _embed_attn(xp, params)
    t_flat = t.reshape(R, EMBED)

    noise1 = jax.random.normal(jax.random.key(1), (BATCH, NTOK, NEXP),
                               dtype=_F32).reshape(R, NEXP)
    noise2 = jax.random.normal(jax.random.key(2), (BATCH, NTOK, NEXP),
                               dtype=_F32).reshape(R, NEXP)

    xln1, gate1 = _router(t_flat, params['moe1'], params['g2'],
                          params['bln2'], noise1)
    xln2, gate2 = _router(t_flat, params['moe2'], params['g3'],
                          params['bln3'], noise2)

    fa1, fb1, ep1 = _moe_sparse(xln1, gate1, params['moe1'])
    fa2, fb2, ep2 = _moe_sparse(xln2, gate2, params['moe2'])
    f1_pad, f2_pad = _combine(fa1, fb1, fa2, fb2)

    f1 = f1_pad[:R].reshape(BATCH, NTOK, EMBED)
    f2 = f2_pad[:R].reshape(BATCH, NTOK, EMBED)
    e1 = ep1[:NEXP * R].reshape(NEXP, BATCH, NTOK, EMBED)
    e2 = ep2[:NEXP * R].reshape(NEXP, BATCH, NTOK, EMBED)
    gt1 = gate1.reshape(BATCH, NTOK, NEXP)
    gt2 = gate2.reshape(BATCH, NTOK, NEXP)

    feat, cls = _head(f2, params['Wc'], params['bc'])
    return (f1, f2, feat, cls, e1, e2, gt1, gt2)


# speedup vs baseline: 5.7857x; 5.7857x over previous
import functools

import jax
import jax.numpy as jnp
from jax import lax
from jax.experimental import pallas as pl
from jax.experimental.pallas import tpu as pltpu
from jax.experimental.pallas import tpu_sc as plsc

IMG = 224; PATCH = 16; C_IN = 3; EMBED = 512; NEXP = 10; TOPK = 2; NHEAD = 8; BATCH = 8
NTOK = (IMG // PATCH) ** 2
PDIM = PATCH * PATCH * C_IN
HDIM = 4 * EMBED
HD = EMBED // NHEAD
R = BATCH * NTOK
TTILE = 224
NTILE = R // TTILE

A = R * TOPK
TILE = 128
GTOT = A // TILE + NEXP
NPAD = GTOT * TILE
FPAD = 1792
EPAD = 16384
NSC = 32

_F32 = jnp.float32


def _dot(a, b, dims):
    return jax.lax.dot_general(a, b, (dims, ((), ())),
                               preferred_element_type=_F32)


def _ln_rows(x, g, b, eps=1e-5):
    m = jnp.mean(x, axis=-1, keepdims=True)
    v = jnp.mean((x - m) ** 2, axis=-1, keepdims=True)
    return (x - m) / jnp.sqrt(v + eps) * g + b


def _embed_attn_body(xp_ref, wp_ref, bp_ref, g1_ref, b1_ref,
                     wq_ref, wk_ref, wv_ref, wo_ref, bo_ref, pos_ref, t_ref):
    x = xp_ref[0]
    t0 = _dot(x, wp_ref[...], ((1,), (1,))) + bp_ref[...]
    ln = _ln_rows(t0, g1_ref[...], b1_ref[...])
    q = _dot(ln, wq_ref[...], ((1,), (1,)))
    k = _dot(ln, wk_ref[...], ((1,), (1,)))
    v = _dot(ln, wv_ref[...], ((1,), (1,)))
    heads = []
    for h in range(NHEAD):
        sl = slice(h * HD, (h + 1) * HD)
        att = _dot(q[:, sl], k[:, sl], ((1,), (1,))) * (HD ** -0.5)
        att = jax.nn.softmax(att, axis=-1)
        heads.append(_dot(att, v[:, sl], ((1,), (0,))))
    o = jnp.concatenate(heads, axis=-1)
    o = _dot(o, wo_ref[...], ((1,), (1,))) + bo_ref[...]
    t_ref[0] = t0 + o + pos_ref[0]


def _embed_attn(xp, p):
    return pl.pallas_call(
        _embed_attn_body,
        grid=(BATCH,),
        in_specs=[
            pl.BlockSpec((1, NTOK, PDIM), lambda b: (b, 0, 0)),
            pl.BlockSpec((EMBED, PDIM), lambda b: (0, 0)),
            pl.BlockSpec((1, EMBED), lambda b: (0, 0)),
            pl.BlockSpec((1, EMBED), lambda b: (0, 0)),
            pl.BlockSpec((1, EMBED), lambda b: (0, 0)),
            pl.BlockSpec((EMBED, EMBED), lambda b: (0, 0)),
            pl.BlockSpec((EMBED, EMBED), lambda b: (0, 0)),
            pl.BlockSpec((EMBED, EMBED), lambda b: (0, 0)),
            pl.BlockSpec((EMBED, EMBED), lambda b: (0, 0)),
            pl.BlockSpec((1, EMBED), lambda b: (0, 0)),
            pl.BlockSpec((1, NTOK, EMBED), lambda b: (0, 0, 0)),
        ],
        out_specs=pl.BlockSpec((1, NTOK, EMBED), lambda b: (b, 0, 0)),
        out_shape=jax.ShapeDtypeStruct((BATCH, NTOK, EMBED), _F32),
    )(xp, p['Wp'], p['bp'].reshape(1, -1), p['g1'].reshape(1, -1),
      p['bln1'].reshape(1, -1), p['Wq'], p['Wk'], p['Wv'], p['Wo'],
      p['bo'].reshape(1, -1), p['pos'])


def _router_body(t_ref, g_ref, b_ref, wt_ref, bt_ref, wn_ref, bn_ref,
                 noise_ref, xln_ref, gate_ref):
    x = _ln_rows(t_ref[...], g_ref[...], b_ref[...])
    logits = _dot(x, wt_ref[...], ((1,), (1,))) + bt_ref[...]
    nl = _dot(x, wn_ref[...], ((1,), (1,))) + bn_ref[...]
    noisy = logits + noise_ref[...] * jax.nn.softplus(nl)
    m1 = jnp.max(noisy, axis=-1, keepdims=True)
    ninf = jnp.float32(-jnp.inf)
    m2 = jnp.max(jnp.where(noisy == m1, ninf, noisy), axis=-1, keepdims=True)
    sel = noisy >= m2
    e = jnp.where(sel, jnp.exp(noisy - m1), 0.0)
    gate_ref[...] = e / jnp.sum(e, axis=-1, keepdims=True)
    xln_ref[...] = x


def _router(t_flat, mp, g, b, noise):
    full = lambda *s: pl.BlockSpec(s, lambda: tuple(0 for _ in s))
    return pl.pallas_call(
        _router_body,
        in_specs=[
            full(R, EMBED), full(1, EMBED), full(1, EMBED),
            full(NEXP, EMBED), full(1, NEXP),
            full(NEXP, EMBED), full(1, NEXP), full(R, NEXP),
        ],
        out_specs=[full(R, EMBED), full(R, NEXP)],
        out_shape=[jax.ShapeDtypeStruct((R, EMBED), _F32),
                   jax.ShapeDtypeStruct((R, NEXP), _F32)],
    )(t_flat, g.reshape(1, -1), b.reshape(1, -1),
      mp['Wt'], mp['bt'].reshape(1, -1), mp['Wn'], mp['bn'].reshape(1, -1),
      noise)


def _route_meta(topi, gating):
    i32 = jnp.int32
    token = (jnp.arange(A, dtype=i32) // TOPK)
    expert = topi.reshape(A).astype(i32)
    key = expert * R + token
    order = jnp.argsort(key)
    e_s, t_s, key_s = expert[order], token[order], key[order]
    counts = jnp.zeros((NEXP,), i32).at[expert].add(1)
    tiles = jnp.maximum((counts + TILE - 1) // TILE, 1)
    cum_tiles = jnp.cumsum(tiles)
    rowstart = (cum_tiles - tiles) * TILE
    gfirst = jnp.cumsum(counts) - counts
    pos_s = rowstart[e_s] + jnp.arange(A, dtype=i32) - gfirst[e_s]
    row_token = (jnp.arange(NPAD, dtype=i32) % R).at[pos_s].set(t_s)
    row_gate = jnp.zeros((NPAD,), _F32).at[pos_s].set(gating[t_s, e_s])
    covered = jnp.zeros((NPAD,), i32).at[pos_s].set(1)
    pad_rows = jnp.argsort(covered)[:NPAD - A].astype(i32)
    tile_expert = jnp.minimum(
        jnp.searchsorted(cum_tiles, jnp.arange(GTOT, dtype=i32), side='right'),
        NEXP - 1).astype(i32)
    first_tile = cum_tiles - tiles
    pos_a = jnp.zeros((A,), i32).at[order].set(pos_s)
    pos0p = jnp.concatenate([pos_a[0::2], pad_rows[:FPAD - R]])
    pos1p = jnp.concatenate([pos_a[1::2], pad_rows[FPAD - R:2 * (FPAD - R)]])
    return row_token, row_gate, tile_expert, first_tile, covered, pos0p, pos1p


def _sc_gather(table, idx, n_rows):
    d = table.shape[1]
    bpw = n_rows // NSC
    mesh = plsc.VectorSubcoreMesh(core_axis_name="c", subcore_axis_name="s")

    @functools.partial(
        pl.kernel, mesh=mesh,
        out_type=jax.ShapeDtypeStruct((n_rows, d), _F32),
        scratch_types=[pltpu.VMEM((bpw,), jnp.int32),
                       pltpu.VMEM((bpw, d), _F32),
                       pltpu.SemaphoreType.DMA],
    )
    def k(table_hbm, idx_hbm, out_hbm, idx_v, rows_v, sem):
        wid = lax.axis_index("s") * 2 + lax.axis_index("c")
        base = wid * bpw
        pltpu.sync_copy(idx_hbm.at[pl.ds(base, bpw)], idx_v)
        pltpu.async_copy(table_hbm.at[idx_v], rows_v, sem).wait()
        pltpu.sync_copy(rows_v, out_hbm.at[pl.ds(base, bpw)])

    return k(table, idx)


def _ffn_body(se_ref, first_ref, rt_ref, cov_ref,
              x_ref, w1_ref, b1_ref, w2_ref, b2_ref, g_ref, o_ref, eo_ref):
    g = pl.program_id(0)
    x = x_ref[...]
    h = jnp.maximum(_dot(x, w1_ref[0], ((1,), (1,))) + b1_ref[0], 0.0)
    o = _dot(h, w2_ref[0], ((1,), (1,))) + b2_ref[0]
    w = o * g_ref[0, 0][:, None]
    o_ref[...] = w

    @pl.when(first_ref[se_ref[g]] == g)
    def _():
        eo_ref[0] = jnp.zeros((R, EMBED), _F32)
    for r in range(TILE):
        @pl.when(cov_ref[g * TILE + r] > 0)
        def _():
            tok = rt_ref[g * TILE + r]
            eo_ref[0, pl.ds(tok, 1), :] = w[r:r + 1, :]


def _moe_ffn(xs, tile_expert, first_tile, row_token, covered, row_gate, mp):
    grid_spec = pltpu.PrefetchScalarGridSpec(
        num_scalar_prefetch=4,
        grid=(GTOT,),
        in_specs=[
            pl.BlockSpec((TILE, EMBED), lambda g, se, ft, rt, cv: (g, 0)),
            pl.BlockSpec((1, HDIM, EMBED),
                         lambda g, se, ft, rt, cv: (se[g], 0, 0)),
            pl.BlockSpec((1, 1, HDIM), lambda g, se, ft, rt, cv: (se[g], 0, 0)),
            pl.BlockSpec((1, EMBED, HDIM),
                         lambda g, se, ft, rt, cv: (se[g], 0, 0)),
            pl.BlockSpec((1, 1, EMBED), lambda g, se, ft, rt, cv: (se[g], 0, 0)),
            pl.BlockSpec((1, 1, TILE), lambda g, se, ft, rt, cv: (g, 0, 0)),
        ],
        out_specs=[
            pl.BlockSpec((TILE, EMBED), lambda g, se, ft, rt, cv: (g, 0)),
            pl.BlockSpec((1, R, EMBED), lambda g, se, ft, rt, cv: (se[g], 0, 0)),
        ],
    )
    return pl.pallas_call(
        _ffn_body,
        grid_spec=grid_spec,
        out_shape=[jax.ShapeDtypeStruct((NPAD, EMBED), _F32),
                   jax.ShapeDtypeStruct((NEXP, R, EMBED), _F32)],
    )(tile_expert, first_tile, row_token, covered,
      xs, mp['W1'], mp['b1'].reshape(NEXP, 1, HDIM),
      mp['W2'], mp['b2'].reshape(NEXP, 1, EMBED),
      row_gate.reshape(GTOT, 1, TILE))


def _moe_sparse(xln, gating, mp):
    topi = lax.top_k(gating, TOPK)[1]
    (row_token, row_gate, tile_expert, first_tile, covered,
     pos0p, pos1p) = _route_meta(topi, gating)
    xs = _sc_gather(xln, row_token, NPAD)
    wrows, e_out = _moe_ffn(xs, tile_expert, first_tile, row_token, covered,
                            row_gate, mp)
    f_ab = _sc_gather(wrows, jnp.concatenate([pos0p, pos1p]), 2 * FPAD)
    return f_ab[:FPAD], f_ab[FPAD:], e_out


def _combine_body(a1_ref, b1_ref, a2_ref, b2_ref, f1_ref, f2_ref):
    f1_ref[...] = a1_ref[...] + b1_ref[...]
    f2_ref[...] = a2_ref[...] + b2_ref[...]


def _combine(a1, b1, a2, b2):
    full = lambda *s: pl.BlockSpec(s, lambda: tuple(0 for _ in s))
    return pl.pallas_call(
        _combine_body,
        in_specs=[full(FPAD, EMBED)] * 4,
        out_specs=[full(FPAD, EMBED)] * 2,
        out_shape=[jax.ShapeDtypeStruct((FPAD, EMBED), _F32)] * 2,
    )(a1, b1, a2, b2)


def _head_body(f2_ref, wc_ref, bc_ref, feat_ref, cls_ref):
    feat = jnp.mean(f2_ref[...], axis=1)
    feat_ref[...] = feat
    cls_ref[...] = _dot(feat, wc_ref[...], ((1,), (1,))) + bc_ref[...]


def _head(f2, wc, bc):
    full = lambda *s: pl.BlockSpec(s, lambda: tuple(0 for _ in s))
    return pl.pallas_call(
        _head_body,
        in_specs=[full(BATCH, NTOK, EMBED), full(NEXP, EMBED), full(1, NEXP)],
        out_specs=[full(BATCH, EMBED), full(BATCH, NEXP)],
        out_shape=[jax.ShapeDtypeStruct((BATCH, EMBED), _F32),
                   jax.ShapeDtypeStruct((BATCH, NEXP), _F32)],
    )(f2, wc, bc.reshape(1, -1))


def kernel(x, params):
    b, c, h, w = x.shape
    xp = x.reshape(b, c, h // PATCH, PATCH, w // PATCH, PATCH)
    xp = xp.transpose(0, 1, 2, 4, 3, 5).reshape(b, c, -1, PATCH * PATCH)
    xp = xp.transpose(0, 2, 1, 3).reshape(b, -1, PDIM)

    t = _embed_attn(xp, params)
    t_flat = t.reshape(R, EMBED)

    noise1 = jax.random.normal(jax.random.key(1), (BATCH, NTOK, NEXP),
                               dtype=_F32).reshape(R, NEXP)
    noise2 = jax.random.normal(jax.random.key(2), (BATCH, NTOK, NEXP),
                               dtype=_F32).reshape(R, NEXP)

    xln1, gate1 = _router(t_flat, params['moe1'], params['g2'],
                          params['bln2'], noise1)
    xln2, gate2 = _router(t_flat, params['moe2'], params['g3'],
                          params['bln3'], noise2)

    fa1, fb1, ep1 = _moe_sparse(xln1, gate1, params['moe1'])
    fa2, fb2, ep2 = _moe_sparse(xln2, gate2, params['moe2'])
    f1_pad, f2_pad = _combine(fa1, fb1, fa2, fb2)

    f1 = f1_pad[:R].reshape(BATCH, NTOK, EMBED)
    f2 = f2_pad[:R].reshape(BATCH, NTOK, EMBED)
    e1 = ep1.reshape(NEXP, BATCH, NTOK, EMBED)
    e2 = ep2.reshape(NEXP, BATCH, NTOK, EMBED)
    gt1 = gate1.reshape(BATCH, NTOK, NEXP)
    gt2 = gate2.reshape(BATCH, NTOK, NEXP)

    feat, cls = _head(f2, params['Wc'], params['bc'])
    return (f1, f2, feat, cls, e1, e2, gt1, gt2)


# speedup vs baseline: 6.2384x; 1.0783x over previous
import functools

import jax
import jax.numpy as jnp
from jax import lax
from jax.experimental import pallas as pl
from jax.experimental.pallas import tpu as pltpu
from jax.experimental.pallas import tpu_sc as plsc

IMG = 224; PATCH = 16; C_IN = 3; EMBED = 512; NEXP = 10; TOPK = 2; NHEAD = 8; BATCH = 8
NTOK = (IMG // PATCH) ** 2
PDIM = PATCH * PATCH * C_IN
HDIM = 4 * EMBED
HD = EMBED // NHEAD
R = BATCH * NTOK
TTILE = 224
NTILE = R // TTILE

A = R * TOPK
TILE = 128
GTOT = A // TILE + NEXP
NPAD = GTOT * TILE
FPAD = 1792
EPAD = 16384
NSC = 32

_F32 = jnp.float32


def _dot(a, b, dims):
    return jax.lax.dot_general(a, b, (dims, ((), ())),
                               preferred_element_type=_F32)


def _ln_rows(x, g, b, eps=1e-5):
    m = jnp.mean(x, axis=-1, keepdims=True)
    v = jnp.mean((x - m) ** 2, axis=-1, keepdims=True)
    return (x - m) / jnp.sqrt(v + eps) * g + b


def _embed_attn_body(xp_ref, wp_ref, bp_ref, g1_ref, b1_ref,
                     wq_ref, wk_ref, wv_ref, wo_ref, bo_ref, pos_ref, t_ref):
    x = xp_ref[0]
    t0 = _dot(x, wp_ref[...], ((1,), (1,))) + bp_ref[...]
    ln = _ln_rows(t0, g1_ref[...], b1_ref[...])
    q = _dot(ln, wq_ref[...], ((1,), (1,)))
    k = _dot(ln, wk_ref[...], ((1,), (1,)))
    v = _dot(ln, wv_ref[...], ((1,), (1,)))
    heads = []
    for h in range(NHEAD):
        sl = slice(h * HD, (h + 1) * HD)
        att = _dot(q[:, sl], k[:, sl], ((1,), (1,))) * (HD ** -0.5)
        att = jax.nn.softmax(att, axis=-1)
        heads.append(_dot(att, v[:, sl], ((1,), (0,))))
    o = jnp.concatenate(heads, axis=-1)
    o = _dot(o, wo_ref[...], ((1,), (1,))) + bo_ref[...]
    t_ref[0] = t0 + o + pos_ref[0]


def _embed_attn(xp, p):
    return pl.pallas_call(
        _embed_attn_body,
        grid=(BATCH,),
        in_specs=[
            pl.BlockSpec((1, NTOK, PDIM), lambda b: (b, 0, 0)),
            pl.BlockSpec((EMBED, PDIM), lambda b: (0, 0)),
            pl.BlockSpec((1, EMBED), lambda b: (0, 0)),
            pl.BlockSpec((1, EMBED), lambda b: (0, 0)),
            pl.BlockSpec((1, EMBED), lambda b: (0, 0)),
            pl.BlockSpec((EMBED, EMBED), lambda b: (0, 0)),
            pl.BlockSpec((EMBED, EMBED), lambda b: (0, 0)),
            pl.BlockSpec((EMBED, EMBED), lambda b: (0, 0)),
            pl.BlockSpec((EMBED, EMBED), lambda b: (0, 0)),
            pl.BlockSpec((1, EMBED), lambda b: (0, 0)),
            pl.BlockSpec((1, NTOK, EMBED), lambda b: (0, 0, 0)),
        ],
        out_specs=pl.BlockSpec((1, NTOK, EMBED), lambda b: (b, 0, 0)),
        out_shape=jax.ShapeDtypeStruct((BATCH, NTOK, EMBED), _F32),
    )(xp, p['Wp'], p['bp'].reshape(1, -1), p['g1'].reshape(1, -1),
      p['bln1'].reshape(1, -1), p['Wq'], p['Wk'], p['Wv'], p['Wo'],
      p['bo'].reshape(1, -1), p['pos'])


def _router_body(t_ref, g_ref, b_ref, wt_ref, bt_ref, wn_ref, bn_ref,
                 noise_ref, xln_ref, gate_ref):
    x = _ln_rows(t_ref[...], g_ref[...], b_ref[...])
    logits = _dot(x, wt_ref[...], ((1,), (1,))) + bt_ref[...]
    nl = _dot(x, wn_ref[...], ((1,), (1,))) + bn_ref[...]
    noisy = logits + noise_ref[...] * jax.nn.softplus(nl)
    m1 = jnp.max(noisy, axis=-1, keepdims=True)
    ninf = jnp.float32(-jnp.inf)
    m2 = jnp.max(jnp.where(noisy == m1, ninf, noisy), axis=-1, keepdims=True)
    sel = noisy >= m2
    e = jnp.where(sel, jnp.exp(noisy - m1), 0.0)
    gate_ref[...] = e / jnp.sum(e, axis=-1, keepdims=True)
    xln_ref[...] = x


def _router(t_flat, mp, g, b, noise):
    full = lambda *s: pl.BlockSpec(s, lambda: tuple(0 for _ in s))
    return pl.pallas_call(
        _router_body,
        in_specs=[
            full(R, EMBED), full(1, EMBED), full(1, EMBED),
            full(NEXP, EMBED), full(1, NEXP),
            full(NEXP, EMBED), full(1, NEXP), full(R, NEXP),
        ],
        out_specs=[full(R, EMBED), full(R, NEXP)],
        out_shape=[jax.ShapeDtypeStruct((R, EMBED), _F32),
                   jax.ShapeDtypeStruct((R, NEXP), _F32)],
    )(t_flat, g.reshape(1, -1), b.reshape(1, -1),
      mp['Wt'], mp['bt'].reshape(1, -1), mp['Wn'], mp['bn'].reshape(1, -1),
      noise)


def _route_meta(topi, gating):
    i32 = jnp.int32
    expert = topi.astype(i32)
    ef = expert.reshape(A)
    onehot = (ef[:, None] == jnp.arange(NEXP, dtype=i32)[None, :]).astype(i32)
    prefix = jnp.cumsum(onehot, axis=0)
    counts = prefix[-1]
    rank = jnp.take_along_axis(prefix, ef[:, None], axis=1)[:, 0] - 1
    tiles = jnp.maximum((counts + TILE - 1) // TILE, 1)
    cum_tiles = jnp.cumsum(tiles)
    rowstart = (cum_tiles - tiles) * TILE
    pos_a = rowstart[ef] + rank
    token = jnp.arange(A, dtype=i32) // TOPK
    row_token = (jnp.arange(NPAD, dtype=i32) % R).at[pos_a].set(token)
    gsel = jnp.take_along_axis(gating, expert, axis=1).reshape(A)
    row_gate = jnp.zeros((NPAD,), _F32).at[pos_a].set(gsel)
    covered = jnp.zeros((NPAD,), i32).at[pos_a].set(1)
    pad_rows = jnp.nonzero(covered == 0, size=NPAD - A)[0].astype(i32)
    tile_expert = jnp.minimum(
        jnp.searchsorted(cum_tiles, jnp.arange(GTOT, dtype=i32), side='right'),
        NEXP - 1).astype(i32)
    first_tile = cum_tiles - tiles
    pos2 = pos_a.reshape(R, TOPK)
    pos0p = jnp.concatenate([pos2[:, 0], pad_rows[:FPAD - R]])
    pos1p = jnp.concatenate([pos2[:, 1], pad_rows[FPAD - R:2 * (FPAD - R)]])
    return row_token, row_gate, tile_expert, first_tile, covered, pos0p, pos1p


def _sc_gather(table, idx, n_rows):
    d = table.shape[1]
    bpw = n_rows // NSC
    mesh = plsc.VectorSubcoreMesh(core_axis_name="c", subcore_axis_name="s")

    @functools.partial(
        pl.kernel, mesh=mesh,
        out_type=jax.ShapeDtypeStruct((n_rows, d), _F32),
        scratch_types=[pltpu.VMEM((bpw,), jnp.int32),
                       pltpu.VMEM((bpw, d), _F32),
                       pltpu.SemaphoreType.DMA],
    )
    def k(table_hbm, idx_hbm, out_hbm, idx_v, rows_v, sem):
        wid = lax.axis_index("s") * 2 + lax.axis_index("c")
        base = wid * bpw
        pltpu.sync_copy(idx_hbm.at[pl.ds(base, bpw)], idx_v)
        pltpu.async_copy(table_hbm.at[idx_v], rows_v, sem).wait()
        pltpu.sync_copy(rows_v, out_hbm.at[pl.ds(base, bpw)])

    return k(table, idx)


def _ffn_body(se_ref, first_ref, rt_ref, cov_ref,
              x_ref, w1_ref, b1_ref, w2_ref, b2_ref, g_ref, o_ref, eo_ref):
    g = pl.program_id(0)
    x = x_ref[...]
    h = jnp.maximum(_dot(x, w1_ref[0], ((1,), (1,))) + b1_ref[0], 0.0)
    o = _dot(h, w2_ref[0], ((1,), (1,))) + b2_ref[0]
    w = o * g_ref[0, 0][:, None]
    o_ref[...] = w

    @pl.when(first_ref[se_ref[g]] == g)
    def _():
        eo_ref[0] = jnp.zeros((R, EMBED), _F32)
    for r in range(TILE):
        @pl.when(cov_ref[g * TILE + r] > 0)
        def _():
            tok = rt_ref[g * TILE + r]
            eo_ref[0, pl.ds(tok, 1), :] = w[r:r + 1, :]


def _moe_ffn(xs, tile_expert, first_tile, row_token, covered, row_gate, mp):
    grid_spec = pltpu.PrefetchScalarGridSpec(
        num_scalar_prefetch=4,
        grid=(GTOT,),
        in_specs=[
            pl.BlockSpec((TILE, EMBED), lambda g, se, ft, rt, cv: (g, 0)),
            pl.BlockSpec((1, HDIM, EMBED),
                         lambda g, se, ft, rt, cv: (se[g], 0, 0)),
            pl.BlockSpec((1, 1, HDIM), lambda g, se, ft, rt, cv: (se[g], 0, 0)),
            pl.BlockSpec((1, EMBED, HDIM),
                         lambda g, se, ft, rt, cv: (se[g], 0, 0)),
            pl.BlockSpec((1, 1, EMBED), lambda g, se, ft, rt, cv: (se[g], 0, 0)),
            pl.BlockSpec((1, 1, TILE), lambda g, se, ft, rt, cv: (g, 0, 0)),
        ],
        out_specs=[
            pl.BlockSpec((TILE, EMBED), lambda g, se, ft, rt, cv: (g, 0)),
            pl.BlockSpec((1, R, EMBED), lambda g, se, ft, rt, cv: (se[g], 0, 0)),
        ],
    )
    return pl.pallas_call(
        _ffn_body,
        grid_spec=grid_spec,
        out_shape=[jax.ShapeDtypeStruct((NPAD, EMBED), _F32),
                   jax.ShapeDtypeStruct((NEXP, R, EMBED), _F32)],
    )(tile_expert, first_tile, row_token, covered,
      xs, mp['W1'], mp['b1'].reshape(NEXP, 1, HDIM),
      mp['W2'], mp['b2'].reshape(NEXP, 1, EMBED),
      row_gate.reshape(GTOT, 1, TILE))


def _moe_sparse(xln, gating, mp):
    topi = lax.top_k(gating, TOPK)[1]
    (row_token, row_gate, tile_expert, first_tile, covered,
     pos0p, pos1p) = _route_meta(topi, gating)
    xs = _sc_gather(xln, row_token, NPAD)
    wrows, e_out = _moe_ffn(xs, tile_expert, first_tile, row_token, covered,
                            row_gate, mp)
    f_ab = _sc_gather(wrows, jnp.concatenate([pos0p, pos1p]), 2 * FPAD)
    return f_ab[:FPAD], f_ab[FPAD:], e_out


def _combine_body(a1_ref, b1_ref, a2_ref, b2_ref, f1_ref, f2_ref):
    f1_ref[...] = a1_ref[...] + b1_ref[...]
    f2_ref[...] = a2_ref[...] + b2_ref[...]


def _combine(a1, b1, a2, b2):
    full = lambda *s: pl.BlockSpec(s, lambda: tuple(0 for _ in s))
    return pl.pallas_call(
        _combine_body,
        in_specs=[full(FPAD, EMBED)] * 4,
        out_specs=[full(FPAD, EMBED)] * 2,
        out_shape=[jax.ShapeDtypeStruct((FPAD, EMBED), _F32)] * 2,
    )(a1, b1, a2, b2)


def _head_body(f2_ref, wc_ref, bc_ref, feat_ref, cls_ref):
    feat = jnp.mean(f2_ref[...], axis=1)
    feat_ref[...] = feat
    cls_ref[...] = _dot(feat, wc_ref[...], ((1,), (1,))) + bc_ref[...]


def _head(f2, wc, bc):
    full = lambda *s: pl.BlockSpec(s, lambda: tuple(0 for _ in s))
    return pl.pallas_call(
        _head_body,
        in_specs=[full(BATCH, NTOK, EMBED), full(NEXP, EMBED), full(1, NEXP)],
        out_specs=[full(BATCH, EMBED), full(BATCH, NEXP)],
        out_shape=[jax.ShapeDtypeStruct((BATCH, EMBED), _F32),
                   jax.ShapeDtypeStruct((BATCH, NEXP), _F32)],
    )(f2, wc, bc.reshape(1, -1))


def kernel(x, params):
    b, c, h, w = x.shape
    xp = x.reshape(b, c, h // PATCH, PATCH, w // PATCH, PATCH)
    xp = xp.transpose(0, 1, 2, 4, 3, 5).reshape(b, c, -1, PATCH * PATCH)
    xp = xp.transpose(0, 2, 1, 3).reshape(b, -1, PDIM)

    t = _embed_attn(xp, params)
    t_flat = t.reshape(R, EMBED)

    noise1 = jax.random.normal(jax.random.key(1), (BATCH, NTOK, NEXP),
                               dtype=_F32).reshape(R, NEXP)
    noise2 = jax.random.normal(jax.random.key(2), (BATCH, NTOK, NEXP),
                               dtype=_F32).reshape(R, NEXP)

    xln1, gate1 = _router(t_flat, params['moe1'], params['g2'],
                          params['bln2'], noise1)
    xln2, gate2 = _router(t_flat, params['moe2'], params['g3'],
                          params['bln3'], noise2)

    fa1, fb1, ep1 = _moe_sparse(xln1, gate1, params['moe1'])
    fa2, fb2, ep2 = _moe_sparse(xln2, gate2, params['moe2'])
    f1_pad, f2_pad = _combine(fa1, fb1, fa2, fb2)

    f1 = f1_pad[:R].reshape(BATCH, NTOK, EMBED)
    f2 = f2_pad[:R].reshape(BATCH, NTOK, EMBED)
    e1 = ep1.reshape(NEXP, BATCH, NTOK, EMBED)
    e2 = ep2.reshape(NEXP, BATCH, NTOK, EMBED)
    gt1 = gate1.reshape(BATCH, NTOK, NEXP)
    gt2 = gate2.reshape(BATCH, NTOK, NEXP)

    feat, cls = _head(f2, params['Wc'], params['bc'])
    return (f1, f2, feat, cls, e1, e2, gt1, gt2)


# speedup vs baseline: 6.2842x; 1.0073x over previous
import functools

import jax
import jax.numpy as jnp
from jax import lax
from jax.experimental import pallas as pl
from jax.experimental.pallas import tpu as pltpu
from jax.experimental.pallas import tpu_sc as plsc

IMG = 224; PATCH = 16; C_IN = 3; EMBED = 512; NEXP = 10; TOPK = 2; NHEAD = 8; BATCH = 8
NTOK = (IMG // PATCH) ** 2
PDIM = PATCH * PATCH * C_IN
HDIM = 4 * EMBED
HD = EMBED // NHEAD
R = BATCH * NTOK
TTILE = 224
NTILE = R // TTILE

A = R * TOPK
TILE = 128
GTOT = A // TILE + NEXP
NPAD = GTOT * TILE
FPAD = 1792
EPAD = 16384
NSC = 32

_F32 = jnp.float32


def _dot(a, b, dims):
    return jax.lax.dot_general(a, b, (dims, ((), ())),
                               preferred_element_type=_F32)


def _ln_rows(x, g, b, eps=1e-5):
    m = jnp.mean(x, axis=-1, keepdims=True)
    v = jnp.mean((x - m) ** 2, axis=-1, keepdims=True)
    return (x - m) / jnp.sqrt(v + eps) * g + b


def _embed_attn_body(xp_ref, wp_ref, bp_ref, g1_ref, b1_ref,
                     wq_ref, wk_ref, wv_ref, wo_ref, bo_ref, pos_ref, t_ref):
    x = xp_ref[0]
    t0 = _dot(x, wp_ref[...], ((1,), (1,))) + bp_ref[...]
    ln = _ln_rows(t0, g1_ref[...], b1_ref[...])
    q = _dot(ln, wq_ref[...], ((1,), (1,)))
    k = _dot(ln, wk_ref[...], ((1,), (1,)))
    v = _dot(ln, wv_ref[...], ((1,), (1,)))
    heads = []
    for h in range(NHEAD):
        sl = slice(h * HD, (h + 1) * HD)
        att = _dot(q[:, sl], k[:, sl], ((1,), (1,))) * (HD ** -0.5)
        att = jax.nn.softmax(att, axis=-1)
        heads.append(_dot(att, v[:, sl], ((1,), (0,))))
    o = jnp.concatenate(heads, axis=-1)
    o = _dot(o, wo_ref[...], ((1,), (1,))) + bo_ref[...]
    t_ref[0] = t0 + o + pos_ref[0]


def _embed_attn(xp, p):
    return pl.pallas_call(
        _embed_attn_body,
        grid=(BATCH,),
        in_specs=[
            pl.BlockSpec((1, NTOK, PDIM), lambda b: (b, 0, 0)),
            pl.BlockSpec((EMBED, PDIM), lambda b: (0, 0)),
            pl.BlockSpec((1, EMBED), lambda b: (0, 0)),
            pl.BlockSpec((1, EMBED), lambda b: (0, 0)),
            pl.BlockSpec((1, EMBED), lambda b: (0, 0)),
            pl.BlockSpec((EMBED, EMBED), lambda b: (0, 0)),
            pl.BlockSpec((EMBED, EMBED), lambda b: (0, 0)),
            pl.BlockSpec((EMBED, EMBED), lambda b: (0, 0)),
            pl.BlockSpec((EMBED, EMBED), lambda b: (0, 0)),
            pl.BlockSpec((1, EMBED), lambda b: (0, 0)),
            pl.BlockSpec((1, NTOK, EMBED), lambda b: (0, 0, 0)),
        ],
        out_specs=pl.BlockSpec((1, NTOK, EMBED), lambda b: (b, 0, 0)),
        out_shape=jax.ShapeDtypeStruct((BATCH, NTOK, EMBED), _F32),
    )(xp, p['Wp'], p['bp'].reshape(1, -1), p['g1'].reshape(1, -1),
      p['bln1'].reshape(1, -1), p['Wq'], p['Wk'], p['Wv'], p['Wo'],
      p['bo'].reshape(1, -1), p['pos'])


def _router2_body(t_ref,
                  g2_ref, b2_ref, wt1_ref, bt1_ref, wn1_ref, bn1_ref, n1_ref,
                  g3_ref, b3_ref, wt2_ref, bt2_ref, wn2_ref, bn2_ref, n2_ref,
                  xln1_ref, gate1_ref, xln2_ref, gate2_ref):
    t = t_ref[...]

    def route(g_ref, b_ref, wt_ref, bt_ref, wn_ref, bn_ref, noise_ref,
              xln_ref, gate_ref):
        x = _ln_rows(t, g_ref[...], b_ref[...])
        logits = _dot(x, wt_ref[...], ((1,), (1,))) + bt_ref[...]
        nl = _dot(x, wn_ref[...], ((1,), (1,))) + bn_ref[...]
        noisy = logits + noise_ref[...] * jax.nn.softplus(nl)
        m1 = jnp.max(noisy, axis=-1, keepdims=True)
        ninf = jnp.float32(-jnp.inf)
        m2 = jnp.max(jnp.where(noisy == m1, ninf, noisy), axis=-1,
                     keepdims=True)
        sel = noisy >= m2
        e = jnp.where(sel, jnp.exp(noisy - m1), 0.0)
        gate_ref[...] = e / jnp.sum(e, axis=-1, keepdims=True)
        xln_ref[...] = x

    route(g2_ref, b2_ref, wt1_ref, bt1_ref, wn1_ref, bn1_ref, n1_ref,
          xln1_ref, gate1_ref)
    route(g3_ref, b3_ref, wt2_ref, bt2_ref, wn2_ref, bn2_ref, n2_ref,
          xln2_ref, gate2_ref)


def _routers(t_flat, p, noise1, noise2):
    full = lambda *s: pl.BlockSpec(s, lambda: tuple(0 for _ in s))
    mp1, mp2 = p['moe1'], p['moe2']
    per_layer = [full(1, EMBED), full(1, EMBED), full(NEXP, EMBED),
                 full(1, NEXP), full(NEXP, EMBED), full(1, NEXP), full(R, NEXP)]
    return pl.pallas_call(
        _router2_body,
        in_specs=[full(R, EMBED)] + per_layer + per_layer,
        out_specs=[full(R, EMBED), full(R, NEXP)] * 2,
        out_shape=[jax.ShapeDtypeStruct((R, EMBED), _F32),
                   jax.ShapeDtypeStruct((R, NEXP), _F32)] * 2,
    )(t_flat,
      p['g2'].reshape(1, -1), p['bln2'].reshape(1, -1),
      mp1['Wt'], mp1['bt'].reshape(1, -1), mp1['Wn'], mp1['bn'].reshape(1, -1),
      noise1,
      p['g3'].reshape(1, -1), p['bln3'].reshape(1, -1),
      mp2['Wt'], mp2['bt'].reshape(1, -1), mp2['Wn'], mp2['bn'].reshape(1, -1),
      noise2)


def _route_meta(topi, gating):
    i32 = jnp.int32
    expert = topi.astype(i32)
    ef = expert.reshape(A)
    onehot = (ef[:, None] == jnp.arange(NEXP, dtype=i32)[None, :]).astype(i32)
    prefix = jnp.cumsum(onehot, axis=0)
    counts = prefix[-1]
    rank = jnp.take_along_axis(prefix, ef[:, None], axis=1)[:, 0] - 1
    tiles = jnp.maximum((counts + TILE - 1) // TILE, 1)
    cum_tiles = jnp.cumsum(tiles)
    rowstart = (cum_tiles - tiles) * TILE
    pos_a = rowstart[ef] + rank
    token = jnp.arange(A, dtype=i32) // TOPK
    row_token = (jnp.arange(NPAD, dtype=i32) % R).at[pos_a].set(token)
    gsel = jnp.take_along_axis(gating, expert, axis=1).reshape(A)
    row_gate = jnp.zeros((NPAD,), _F32).at[pos_a].set(gsel)
    covered = jnp.zeros((NPAD,), i32).at[pos_a].set(1)
    pad_rows = jnp.nonzero(covered == 0, size=NPAD - A)[0].astype(i32)
    tile_expert = jnp.minimum(
        jnp.searchsorted(cum_tiles, jnp.arange(GTOT, dtype=i32), side='right'),
        NEXP - 1).astype(i32)
    first_tile = cum_tiles - tiles
    pos2 = pos_a.reshape(R, TOPK)
    pos0p = jnp.concatenate([pos2[:, 0], pad_rows[:FPAD - R]])
    pos1p = jnp.concatenate([pos2[:, 1], pad_rows[FPAD - R:2 * (FPAD - R)]])
    return row_token, row_gate, tile_expert, first_tile, covered, pos0p, pos1p


def _sc_gather(table, idx, n_rows):
    d = table.shape[1]
    bpw = n_rows // NSC
    mesh = plsc.VectorSubcoreMesh(core_axis_name="c", subcore_axis_name="s")

    @functools.partial(
        pl.kernel, mesh=mesh,
        out_type=jax.ShapeDtypeStruct((n_rows, d), _F32),
        scratch_types=[pltpu.VMEM((bpw,), jnp.int32),
                       pltpu.VMEM((bpw, d), _F32),
                       pltpu.SemaphoreType.DMA],
    )
    def k(table_hbm, idx_hbm, out_hbm, idx_v, rows_v, sem):
        wid = lax.axis_index("s") * 2 + lax.axis_index("c")
        base = wid * bpw
        pltpu.sync_copy(idx_hbm.at[pl.ds(base, bpw)], idx_v)
        pltpu.async_copy(table_hbm.at[idx_v], rows_v, sem).wait()
        pltpu.sync_copy(rows_v, out_hbm.at[pl.ds(base, bpw)])

    return k(table, idx)


def _ffn_body(se_ref, first_ref, rt_ref, cov_ref,
              x_ref, w1_ref, b1_ref, w2_ref, b2_ref, g_ref, o_ref, eo_ref):
    g = pl.program_id(0)
    x = x_ref[...]
    h = jnp.maximum(_dot(x, w1_ref[0], ((1,), (1,))) + b1_ref[0], 0.0)
    o = _dot(h, w2_ref[0], ((1,), (1,))) + b2_ref[0]
    w = o * g_ref[0, 0][:, None]
    o_ref[...] = w

    @pl.when(first_ref[se_ref[g]] == g)
    def _():
        eo_ref[0] = jnp.zeros((R, EMBED), _F32)
    for r in range(TILE):
        @pl.when(cov_ref[g * TILE + r] > 0)
        def _():
            tok = rt_ref[g * TILE + r]
            eo_ref[0, pl.ds(tok, 1), :] = w[r:r + 1, :]


def _moe_ffn(xs, tile_expert, first_tile, row_token, covered, row_gate, mp):
    grid_spec = pltpu.PrefetchScalarGridSpec(
        num_scalar_prefetch=4,
        grid=(GTOT,),
        in_specs=[
            pl.BlockSpec((TILE, EMBED), lambda g, se, ft, rt, cv: (g, 0)),
            pl.BlockSpec((1, HDIM, EMBED),
                         lambda g, se, ft, rt, cv: (se[g], 0, 0)),
            pl.BlockSpec((1, 1, HDIM), lambda g, se, ft, rt, cv: (se[g], 0, 0)),
            pl.BlockSpec((1, EMBED, HDIM),
                         lambda g, se, ft, rt, cv: (se[g], 0, 0)),
            pl.BlockSpec((1, 1, EMBED), lambda g, se, ft, rt, cv: (se[g], 0, 0)),
            pl.BlockSpec((1, 1, TILE), lambda g, se, ft, rt, cv: (g, 0, 0)),
        ],
        out_specs=[
            pl.BlockSpec((TILE, EMBED), lambda g, se, ft, rt, cv: (g, 0)),
            pl.BlockSpec((1, R, EMBED), lambda g, se, ft, rt, cv: (se[g], 0, 0)),
        ],
    )
    return pl.pallas_call(
        _ffn_body,
        grid_spec=grid_spec,
        out_shape=[jax.ShapeDtypeStruct((NPAD, EMBED), _F32),
                   jax.ShapeDtypeStruct((NEXP, R, EMBED), _F32)],
    )(tile_expert, first_tile, row_token, covered,
      xs, mp['W1'], mp['b1'].reshape(NEXP, 1, HDIM),
      mp['W2'], mp['b2'].reshape(NEXP, 1, EMBED),
      row_gate.reshape(GTOT, 1, TILE))


def _moe_sparse(xln, gating, mp):
    topi = lax.top_k(gating, TOPK)[1]
    (row_token, row_gate, tile_expert, first_tile, covered,
     pos0p, pos1p) = _route_meta(topi, gating)
    xs = _sc_gather(xln, row_token, NPAD)
    wrows, e_out = _moe_ffn(xs, tile_expert, first_tile, row_token, covered,
                            row_gate, mp)
    f_ab = _sc_gather(wrows, jnp.concatenate([pos0p, pos1p]), 2 * FPAD)
    return f_ab[:FPAD], f_ab[FPAD:], e_out


def _head_body(a1_ref, b1_ref, a2_ref, b2_ref, wc_ref, bc_ref,
               f1_ref, f2_ref, feat_ref, cls_ref):
    f1_ref[...] = a1_ref[...] + b1_ref[...]
    f2 = a2_ref[...] + b2_ref[...]
    f2_ref[...] = f2
    feat = jnp.concatenate(
        [jnp.mean(f2[b * NTOK:(b + 1) * NTOK], axis=0, keepdims=True)
         for b in range(BATCH)], axis=0)
    feat_ref[...] = feat
    cls_ref[...] = _dot(feat, wc_ref[...], ((1,), (1,))) + bc_ref[...]


def _head(a1, b1, a2, b2, wc, bc):
    full = lambda *s: pl.BlockSpec(s, lambda: tuple(0 for _ in s))
    return pl.pallas_call(
        _head_body,
        in_specs=[full(FPAD, EMBED)] * 4 + [full(NEXP, EMBED), full(1, NEXP)],
        out_specs=[full(FPAD, EMBED), full(FPAD, EMBED),
                   full(BATCH, EMBED), full(BATCH, NEXP)],
        out_shape=[jax.ShapeDtypeStruct((FPAD, EMBED), _F32),
                   jax.ShapeDtypeStruct((FPAD, EMBED), _F32),
                   jax.ShapeDtypeStruct((BATCH, EMBED), _F32),
                   jax.ShapeDtypeStruct((BATCH, NEXP), _F32)],
    )(a1, b1, a2, b2, wc, bc.reshape(1, -1))


def kernel(x, params):
    b, c, h, w = x.shape
    xp = x.reshape(b, c, h // PATCH, PATCH, w // PATCH, PATCH)
    xp = xp.transpose(0, 1, 2, 4, 3, 5).reshape(b, c, -1, PATCH * PATCH)
    xp = xp.transpose(0, 2, 1, 3).reshape(b, -1, PDIM)

    t = _embed_attn(xp, params)
    t_flat = t.reshape(R, EMBED)

    noise1 = jax.random.normal(jax.random.key(1), (BATCH, NTOK, NEXP),
                               dtype=_F32).reshape(R, NEXP)
    noise2 = jax.random.normal(jax.random.key(2), (BATCH, NTOK, NEXP),
                               dtype=_F32).reshape(R, NEXP)

    xln1, gate1, xln2, gate2 = _routers(t_flat, params, noise1, noise2)

    fa1, fb1, ep1 = _moe_sparse(xln1, gate1, params['moe1'])
    fa2, fb2, ep2 = _moe_sparse(xln2, gate2, params['moe2'])
    f1_pad, f2_pad, feat, cls = _head(fa1, fb1, fa2, fb2,
                                      params['Wc'], params['bc'])

    f1 = f1_pad[:R].reshape(BATCH, NTOK, EMBED)
    f2 = f2_pad[:R].reshape(BATCH, NTOK, EMBED)
    e1 = ep1.reshape(NEXP, BATCH, NTOK, EMBED)
    e2 = ep2.reshape(NEXP, BATCH, NTOK, EMBED)
    gt1 = gate1.reshape(BATCH, NTOK, NEXP)
    gt2 = gate2.reshape(BATCH, NTOK, NEXP)
    return (f1, f2, feat, cls, e1, e2, gt1, gt2)
